# Initial kernel scaffold; baseline (speedup 1.0000x reference)
import jax
import jax.numpy as jnp
from jax.experimental import pallas as pl


def kernel(output, target):
    raise NotImplementedError("write your pallas kernel here")


# speedup vs baseline: 3.2713x; 3.2713x over previous
import functools

import jax
import jax.numpy as jnp
from jax.experimental import pallas as pl
from jax.experimental.pallas import tpu as pltpu


def _body(num_classes, n_total, tgt_ref, x_ref, loss_ref, acc_ref):
    b = pl.program_id(0)
    r = pl.program_id(1)

    @pl.when(jnp.logical_and(b == 0, r == 0))
    def _init():
        acc_ref[...] = jnp.zeros_like(acc_ref)

    x = x_ref[0]
    t = tgt_ref[0]

    m = x[0]
    p = jnp.zeros_like(t)
    for c in range(1, num_classes):
        xc = x[c]
        g = xc > m
        m = jnp.where(g, xc, m)
        p = jnp.where(g, c, p)

    one = jnp.float32(1.0)
    zero = jnp.float32(0.0)
    for c in range(num_classes):
        te = t == c
        pe = p == c
        both = jnp.logical_and(te, pe)
        acc_ref[0, c] += jnp.sum(jnp.where(te, one, zero), axis=0)
        acc_ref[1, c] += jnp.sum(jnp.where(pe, one, zero), axis=0)
        acc_ref[2, c] += jnp.sum(jnp.where(both, one, zero), axis=0)

    last = jnp.logical_and(b == pl.num_programs(0) - 1,
                           r == pl.num_programs(1) - 1)

    @pl.when(last)
    def _finish():
        sums = jnp.sum(acc_ref[...], axis=2)
        ht = sums[0]
        hp = sums[1]
        dg = sums[2]
        smooth = jnp.float32(1e-6)
        ntot = jnp.float32(n_total)
        sens = (dg + smooth) / (hp + smooth)
        spec = (ntot - ht - hp + dg + smooth) / (ntot - hp + smooth)
        mean = jnp.sum(0.5 * sens + 0.5 * spec) / num_classes
        loss_ref[...] = 1.0 - mean.reshape(1, 1)


def kernel(output, target):
    B, C, H, W = output.shape
    Hb = 64
    grid = (B, H // Hb)
    loss = pl.pallas_call(
        functools.partial(_body, C, B * H * W),
        grid=grid,
        in_specs=[
            pl.BlockSpec((1, Hb, W), lambda b, r: (b, r, 0)),
            pl.BlockSpec((1, C, Hb, W), lambda b, r: (b, 0, r, 0)),
        ],
        out_specs=pl.BlockSpec((1, 1), lambda b, r: (0, 0)),
        out_shape=jax.ShapeDtypeStruct((1, 1), jnp.float32),
        scratch_shapes=[pltpu.VMEM((3, C, W), jnp.float32)],
        compiler_params=pltpu.CompilerParams(
            dimension_semantics=("arbitrary", "arbitrary"),
        ),
    )(target, output)
    return loss[0, 0]
